# Initial kernel scaffold; baseline (speedup 1.0000x reference)
import jax
import jax.numpy as jnp
from jax.experimental import pallas as pl


def kernel(x, edge_index, gru_W_ih, gru_W_hh, gru_b_ih, gru_b_hh, conv1_W, conv1_b, conv2_W, conv2_b, mlp_W1, mlp_b1, mlp_W2, mlp_b2):
    raise NotImplementedError("write your pallas kernel here")


# speedup vs baseline: 9.4454x; 9.4454x over previous
import functools

import jax
import jax.numpy as jnp
from jax import lax
from jax.experimental import pallas as pl
from jax.experimental.pallas import tpu as pltpu
from jax.experimental.pallas import tpu_sc as plsc

N = 100000
E = 1600000
SEQ = 8
IN = 16
H = 32

NC = 2
NS = 16
NHALF = N // NC
ACC_ROWS = 50056
TRASH = 50048
TPT = 3128
TPT_LO = 3080

SUB = 128
CHUNK_ROWS = 4
CHUNK = SUB * CHUNK_ROWS
E_PAD = 1605632
EROWS = E_PAD // SUB
CPT_CONV = 196
CPT_MLP = 98

BN = 2000
BE = 8192


def _sc_mesh():
    return plsc.VectorSubcoreMesh(core_axis_name="c", subcore_axis_name="s")


@functools.partial(
    pl.kernel,
    out_type=jax.ShapeDtypeStruct((N,), jnp.float32),
    mesh=_sc_mesh(),
    compiler_params=pltpu.CompilerParams(use_tc_tiling_on_sc=False),
    scratch_types=[
        pltpu.VMEM((CHUNK_ROWS, SUB), jnp.int32),
        pltpu.VMEM((TPT,), jnp.float32),
        pltpu.VMEM_SHARED((ACC_ROWS,), jnp.float32),
        pltpu.SemaphoreType.DMA,
    ],
)
def _deg_sc(dst2, ones_hbm, deg_out, didx, vstage, acc, sem):
    c = lax.axis_index("c")
    s = lax.axis_index("s")
    coff = c * NHALF
    pltpu.sync_copy(ones_hbm.at[pl.ds(0, TPT)], vstage)
    pltpu.sync_copy(vstage, acc.at[pl.ds(s * TPT, TPT)])
    plsc.subcore_barrier()
    ones_v = vstage.at[pl.ds(0, SUB)]

    def chunk(i, _):
        rowbase = (s * CPT_CONV + i) * CHUNK_ROWS
        pltpu.sync_copy(dst2.at[pl.ds(rowbase, CHUNK_ROWS)], didx)

        def localize(j, _):
            r = j >> 3
            q = (j & 7) * 16
            d = didx[r, pl.ds(q, 16)]
            dl = d - coff
            ok = (dl >= 0) & (dl < NHALF)
            didx[r, pl.ds(q, 16)] = jnp.where(ok, dl, TRASH)
            return 0

        lax.fori_loop(0, CHUNK_ROWS * 8, localize, 0)
        for k in range(CHUNK_ROWS):
            pltpu.async_copy(ones_v, acc.at[didx.at[k]], sem, add=True)
        for k in range(CHUNK_ROWS):
            pltpu.make_async_copy(ones_v, acc.at[didx.at[k]], sem).wait()
        return 0

    lax.fori_loop(0, CPT_CONV, chunk, 0)
    plsc.subcore_barrier()
    base = c * NHALF + s * TPT
    pltpu.sync_copy(acc.at[pl.ds(s * TPT, TPT_LO)], vstage.at[pl.ds(0, TPT_LO)])
    pltpu.sync_copy(vstage.at[pl.ds(0, TPT_LO)], deg_out.at[pl.ds(base, TPT_LO)])

    @pl.when(s < NS - 1)
    def _():
        pltpu.sync_copy(acc.at[pl.ds(s * TPT + TPT_LO, TPT - TPT_LO)],
                        vstage.at[pl.ds(0, TPT - TPT_LO)])
        pltpu.sync_copy(vstage.at[pl.ds(0, TPT - TPT_LO)],
                        deg_out.at[pl.ds(base + TPT_LO, TPT - TPT_LO)])


@functools.partial(
    pl.kernel,
    out_type=jax.ShapeDtypeStruct((N, H), jnp.float32),
    mesh=_sc_mesh(),
    compiler_params=pltpu.CompilerParams(use_tc_tiling_on_sc=False),
    scratch_types=[
        pltpu.VMEM((CHUNK_ROWS, SUB), jnp.int32),
        pltpu.VMEM((CHUNK_ROWS, SUB), jnp.int32),
        pltpu.VMEM((CHUNK, H), jnp.float32),
        pltpu.VMEM_SHARED((ACC_ROWS, H), jnp.float32),
        pltpu.SemaphoreType.DMA,
        pltpu.SemaphoreType.DMA,
    ],
)
def _conv_sc(g, src2, dst2, s_out, sidx, didx, rows, acc, gsem, ssem):
    c = lax.axis_index("c")
    s = lax.axis_index("s")
    coff = c * NHALF
    def _stage(src_ref, src_base, dst_ref, dst_base):
        off = 0
        for piece in (CHUNK,) * 6 + (TPT_LO - 6 * CHUNK,):
            pltpu.sync_copy(src_ref.at[pl.ds(src_base + off, piece)],
                            rows.at[pl.ds(0, piece)])
            pltpu.sync_copy(rows.at[pl.ds(0, piece)],
                            dst_ref.at[pl.ds(dst_base + off, piece)])
            off += piece

        @pl.when(s < NS - 1)
        def _():
            pltpu.sync_copy(src_ref.at[pl.ds(src_base + TPT_LO, TPT - TPT_LO)],
                            rows.at[pl.ds(0, TPT - TPT_LO)])
            pltpu.sync_copy(rows.at[pl.ds(0, TPT - TPT_LO)],
                            dst_ref.at[pl.ds(dst_base + TPT_LO, TPT - TPT_LO)])

    _stage(g, coff + s * TPT, acc, s * TPT)
    plsc.subcore_barrier()

    def chunk(i, _):
        rowbase = (s * CPT_CONV + i) * CHUNK_ROWS
        pltpu.sync_copy(src2.at[pl.ds(rowbase, CHUNK_ROWS)], sidx)
        pltpu.sync_copy(dst2.at[pl.ds(rowbase, CHUNK_ROWS)], didx)

        def localize(j, _):
            r = j >> 3
            q = (j & 7) * 16
            d = didx[r, pl.ds(q, 16)]
            dl = d - coff
            ok = (dl >= 0) & (dl < NHALF)
            didx[r, pl.ds(q, 16)] = jnp.where(ok, dl, TRASH)
            return 0

        lax.fori_loop(0, CHUNK_ROWS * 8, localize, 0)
        for k in range(CHUNK_ROWS):
            pltpu.async_copy(g.at[sidx.at[k]], rows.at[pl.ds(k * SUB, SUB)], gsem)
        for k in range(CHUNK_ROWS):
            pltpu.make_async_copy(g.at[sidx.at[k]],
                                  rows.at[pl.ds(k * SUB, SUB)], gsem).wait()
        for k in range(CHUNK_ROWS):
            pltpu.async_copy(rows.at[pl.ds(k * SUB, SUB)], acc.at[didx.at[k]],
                             ssem, add=True)
        for k in range(CHUNK_ROWS):
            pltpu.make_async_copy(rows.at[pl.ds(k * SUB, SUB)],
                                  acc.at[didx.at[k]], ssem).wait()
        return 0

    lax.fori_loop(0, CPT_CONV, chunk, 0)
    plsc.subcore_barrier()
    _stage(acc, s * TPT, s_out, coff + s * TPT)


@functools.partial(
    pl.kernel,
    out_type=jax.ShapeDtypeStruct((E_PAD, H), jnp.float32),
    mesh=_sc_mesh(),
    compiler_params=pltpu.CompilerParams(use_tc_tiling_on_sc=False),
    scratch_types=[
        pltpu.VMEM((CHUNK_ROWS, SUB), jnp.int32),
        pltpu.VMEM((CHUNK_ROWS, SUB), jnp.int32),
        pltpu.VMEM((CHUNK, H), jnp.float32),
        pltpu.SemaphoreType.DMA,
    ],
)
def _mlp_sc(a, cc, src2, dst2, u_out, sidx, didx, rows, sem):
    c = lax.axis_index("c")
    s = lax.axis_index("s")
    wid = c * NS + s

    def chunk(i, _):
        rowbase = (wid * CPT_MLP + i) * CHUNK_ROWS
        pltpu.sync_copy(src2.at[pl.ds(rowbase, CHUNK_ROWS)], sidx)
        pltpu.sync_copy(dst2.at[pl.ds(rowbase, CHUNK_ROWS)], didx)
        for k in range(CHUNK_ROWS):
            pltpu.async_copy(a.at[sidx.at[k]], rows.at[pl.ds(k * SUB, SUB)], sem)
        for k in range(CHUNK_ROWS):
            pltpu.make_async_copy(a.at[sidx.at[k]],
                                  rows.at[pl.ds(k * SUB, SUB)], sem).wait()
        for k in range(CHUNK_ROWS):
            pltpu.async_copy(cc.at[didx.at[k]], rows.at[pl.ds(k * SUB, SUB)],
                             sem, add=True)
        for k in range(CHUNK_ROWS):
            pltpu.make_async_copy(cc.at[didx.at[k]],
                                  rows.at[pl.ds(k * SUB, SUB)], sem).wait()
        pltpu.sync_copy(rows, u_out.at[pl.ds(rowbase * SUB, CHUNK)])
        return 0

    lax.fori_loop(0, CPT_MLP, chunk, 0)


def _gru_body(x_ref, wir, wiz, win, whr, whz, whn, br, bz, bin_, bhn, out_ref):
    blk = x_ref.shape[0]
    h = jnp.zeros((blk, H), jnp.float32)
    f32 = jnp.float32
    for t in range(SEQ):
        xt = x_ref[:, pl.ds(t * IN, IN)]
        r = jax.nn.sigmoid(jnp.dot(xt, wir[...], preferred_element_type=f32)
                           + jnp.dot(h, whr[...], preferred_element_type=f32)
                           + br[...])
        z = jax.nn.sigmoid(jnp.dot(xt, wiz[...], preferred_element_type=f32)
                           + jnp.dot(h, whz[...], preferred_element_type=f32)
                           + bz[...])
        n = jnp.tanh(jnp.dot(xt, win[...], preferred_element_type=f32) + bin_[...]
                     + r * (jnp.dot(h, whn[...], preferred_element_type=f32)
                            + bhn[...]))
        h = (1.0 - z) * n + z * h
    out_ref[...] = h


def _gru_tc(x2, wir, wiz, win, whr, whz, whn, br, bz, bin_, bhn):
    grid = N // BN
    wspec16 = pl.BlockSpec((IN, H), lambda i: (0, 0))
    wspec32 = pl.BlockSpec((H, H), lambda i: (0, 0))
    bspec = pl.BlockSpec((1, H), lambda i: (0, 0))
    return pl.pallas_call(
        _gru_body,
        grid=(grid,),
        in_specs=[pl.BlockSpec((BN, SEQ * IN), lambda i: (i, 0)),
                  wspec16, wspec16, wspec16, wspec32, wspec32, wspec32,
                  bspec, bspec, bspec, bspec],
        out_specs=pl.BlockSpec((BN, H), lambda i: (i, 0)),
        out_shape=jax.ShapeDtypeStruct((N, H), jnp.float32),
    )(x2, wir, wiz, win, whr, whz, whn, br, bz, bin_, bhn)


def _prep_body(deg_ref, h_ref, w_ref, dinv_ref, g_ref):
    dinv = lax.rsqrt(deg_ref[...])
    dinv_ref[...] = dinv
    g_ref[...] = jnp.dot(h_ref[...], w_ref[...],
                         preferred_element_type=jnp.float32) * dinv


def _prep_tc(deg, h, w):
    grid = N // BN
    return pl.pallas_call(
        _prep_body,
        grid=(grid,),
        in_specs=[pl.BlockSpec((BN, 1), lambda i: (i, 0)),
                  pl.BlockSpec((BN, H), lambda i: (i, 0)),
                  pl.BlockSpec((H, H), lambda i: (0, 0))],
        out_specs=[pl.BlockSpec((BN, 1), lambda i: (i, 0)),
                   pl.BlockSpec((BN, H), lambda i: (i, 0))],
        out_shape=[jax.ShapeDtypeStruct((N, 1), jnp.float32),
                   jax.ShapeDtypeStruct((N, H), jnp.float32)],
    )(deg, h, w)


def _mid_body(s_ref, dinv_ref, b_ref, w_ref, g_ref):
    h1 = jax.nn.relu(dinv_ref[...] * s_ref[...] + b_ref[...])
    g_ref[...] = jnp.dot(h1, w_ref[...],
                         preferred_element_type=jnp.float32) * dinv_ref[...]


def _mid_tc(s1, dinv, b, w):
    grid = N // BN
    return pl.pallas_call(
        _mid_body,
        grid=(grid,),
        in_specs=[pl.BlockSpec((BN, H), lambda i: (i, 0)),
                  pl.BlockSpec((BN, 1), lambda i: (i, 0)),
                  pl.BlockSpec((1, H), lambda i: (0, 0)),
                  pl.BlockSpec((H, H), lambda i: (0, 0))],
        out_specs=pl.BlockSpec((BN, H), lambda i: (i, 0)),
        out_shape=jax.ShapeDtypeStruct((N, H), jnp.float32),
    )(s1, dinv, b, w)


def _post_body(s_ref, dinv_ref, b_ref, wa_ref, ba_ref, wc_ref, a_ref, c_ref):
    h2 = jax.nn.relu(dinv_ref[...] * s_ref[...] + b_ref[...])
    a_ref[...] = jnp.dot(h2, wa_ref[...],
                         preferred_element_type=jnp.float32) + ba_ref[...]
    c_ref[...] = jnp.dot(h2, wc_ref[...], preferred_element_type=jnp.float32)


def _post_tc(s2, dinv, b, wa, ba, wc):
    grid = N // BN
    return pl.pallas_call(
        _post_body,
        grid=(grid,),
        in_specs=[pl.BlockSpec((BN, H), lambda i: (i, 0)),
                  pl.BlockSpec((BN, 1), lambda i: (i, 0)),
                  pl.BlockSpec((1, H), lambda i: (0, 0)),
                  pl.BlockSpec((H, H), lambda i: (0, 0)),
                  pl.BlockSpec((1, H), lambda i: (0, 0)),
                  pl.BlockSpec((H, H), lambda i: (0, 0))],
        out_specs=[pl.BlockSpec((BN, H), lambda i: (i, 0)),
                   pl.BlockSpec((BN, H), lambda i: (i, 0))],
        out_shape=[jax.ShapeDtypeStruct((N, H), jnp.float32),
                   jax.ShapeDtypeStruct((N, H), jnp.float32)],
    )(s2, dinv, b, wa, ba, wc)


def _final_body(u_ref, w_ref, b_ref, out_ref):
    out_ref[...] = (jnp.dot(jax.nn.relu(u_ref[...]), w_ref[...],
                            preferred_element_type=jnp.float32) + b_ref[...])


def _final_tc(u, w2, b2):
    grid = E_PAD // BE
    return pl.pallas_call(
        _final_body,
        grid=(grid,),
        in_specs=[pl.BlockSpec((BE, H), lambda i: (i, 0)),
                  pl.BlockSpec((H, 1), lambda i: (0, 0)),
                  pl.BlockSpec((1, 1), lambda i: (0, 0))],
        out_specs=pl.BlockSpec((BE, 1), lambda i: (i, 0)),
        out_shape=jax.ShapeDtypeStruct((E_PAD, 1), jnp.float32),
    )(u, w2, b2)


def kernel(x, edge_index, gru_W_ih, gru_W_hh, gru_b_ih, gru_b_hh,
           conv1_W, conv1_b, conv2_W, conv2_b,
           mlp_W1, mlp_b1, mlp_W2, mlp_b2):
    f32 = jnp.float32
    x2 = x.reshape(N, SEQ * IN)

    wir = gru_W_ih[:H].T
    wiz = gru_W_ih[H:2 * H].T
    win = gru_W_ih[2 * H:].T
    whr = gru_W_hh[:H].T
    whz = gru_W_hh[H:2 * H].T
    whn = gru_W_hh[2 * H:].T
    br = (gru_b_ih[:H] + gru_b_hh[:H]).reshape(1, H)
    bz = (gru_b_ih[H:2 * H] + gru_b_hh[H:2 * H]).reshape(1, H)
    bin_ = gru_b_ih[2 * H:].reshape(1, H)
    bhn = gru_b_hh[2 * H:].reshape(1, H)

    src = edge_index[0]
    dst = edge_index[1]
    pad = E_PAD - E
    src_p = jnp.concatenate([src, jnp.zeros((pad,), jnp.int32)]).reshape(EROWS, SUB)
    dst_conv = jnp.concatenate([dst, jnp.full((pad,), N, jnp.int32)]).reshape(EROWS, SUB)
    dst_mlp = jnp.concatenate([dst, jnp.zeros((pad,), jnp.int32)]).reshape(EROWS, SUB)
    ones_hbm = jnp.ones((TPT,), f32)

    h = _gru_tc(x2, wir, wiz, win, whr, whz, whn, br, bz, bin_, bhn)
    deg = _deg_sc(dst_conv, ones_hbm)
    dinv, g1 = _prep_tc(deg.reshape(N, 1), h, conv1_W)
    s1 = _conv_sc(g1, src_p, dst_conv)
    g2 = _mid_tc(s1, dinv, conv1_b.reshape(1, H), conv2_W)
    s2 = _conv_sc(g2, src_p, dst_conv)
    a, cc = _post_tc(s2, dinv, conv2_b.reshape(1, H),
                     mlp_W1[:H], mlp_b1.reshape(1, H), mlp_W1[H:])
    u = _mlp_sc(a, cc, src_p, dst_mlp)
    logits = _final_tc(u, mlp_W2, mlp_b2.reshape(1, 1))
    return logits[:E, 0]
